# baseline (speedup 1.0000x reference)
import jax
import jax.numpy as jnp
from jax.experimental import pallas as pl
from jax.experimental.pallas import tpu as pltpu

_C = 19
_B = 65536
_EPS = 1e-30


def _owloss_tc_kernel(g_ref, x_ref, mav_ref, pc_ref, out_ref,
                      wb_ref, acc_t, acc_c):
    i = pl.program_id(0)
    nsteps = pl.num_programs(0)

    @pl.when(i == 0)
    def _init():
        acc_t[...] = jnp.zeros_like(acc_t)
        acc_c[...] = jnp.zeros_like(acc_c)
        mav = mav_ref[...]
        mns = jnp.sum(mav * mav, axis=1, keepdims=True)
        w = mav * jax.lax.rsqrt(jnp.maximum(mns, _EPS))
        wb_ref[...] = w.astype(jnp.bfloat16)

    xt = x_ref[...]
    g = g_ref[0]

    ones_row = jnp.ones((1, _C), jnp.bfloat16)
    nsq = jax.lax.dot_general(ones_row, xt * xt, (((1,), (0,)), ((), ())),
                              preferred_element_type=jnp.float32)
    rnl = jax.lax.rsqrt(jnp.maximum(nsq, _EPS)).astype(jnp.bfloat16)
    u = xt * rnl

    lbl = jax.lax.broadcasted_iota(jnp.int32, (_C, 1), 0)
    ohb = (lbl == g).astype(jnp.bfloat16)
    t = jax.lax.dot_general(u, ohb, (((1,), (1,)), ((), ())),
                            preferred_element_type=jnp.float32)
    cnt = jax.lax.dot_general(jnp.ones((1, _B), jnp.bfloat16), ohb,
                              (((1,), (1,)), ((), ())),
                              preferred_element_type=jnp.float32)
    acc_t[...] += t
    acc_c[...] += cnt

    @pl.when(i == nsteps - 1)
    def _finish():
        s = jax.lax.dot_general(wb_ref[...].astype(jnp.float32), acc_t[...],
                                (((1,), (0,)), ((), ())),
                                preferred_element_type=jnp.float32)
        r = jax.lax.broadcasted_iota(jnp.int32, (_C, _C), 0)
        q = jax.lax.broadcasted_iota(jnp.int32, (_C, _C), 1)
        cs = jnp.sum(jnp.where(r == q, s, 0.0), axis=1, keepdims=True)
        cT = jnp.sum(jnp.where(q == lbl, acc_c[...], 0.0),
                     axis=1, keepdims=True)
        pc = pc_ref[...]
        present = cT > 0.0
        minl = jnp.min(jnp.where(present, lbl, _C))
        include = present & (lbl != minl) & (pc > 0.0)
        means = (cT - cs) / jnp.maximum(cT, 1.0)
        terms = jnp.where(include, means, 0.0)
        out_ref[...] = jnp.sum(terms, axis=(0, 1), keepdims=True).reshape(1, 1)


def kernel(logits, sem_gt, is_train, mav_table, prev_count):
    n = logits.shape[0]
    nsteps = n // _B
    xt = logits.astype(jnp.bfloat16).T
    g3 = sem_gt.reshape(nsteps, 1, _B)
    pc2 = prev_count.reshape(_C, 1)
    out = pl.pallas_call(
        _owloss_tc_kernel,
        grid=(nsteps,),
        in_specs=[
            pl.BlockSpec((1, 1, _B), lambda i: (i, 0, 0)),
            pl.BlockSpec((_C, _B), lambda i: (0, i)),
            pl.BlockSpec((_C, _C), lambda i: (0, 0)),
            pl.BlockSpec((_C, 1), lambda i: (0, 0)),
        ],
        out_specs=pl.BlockSpec((1, 1), lambda i: (0, 0)),
        out_shape=jax.ShapeDtypeStruct((1, 1), jnp.float32),
        scratch_shapes=[
            pltpu.VMEM((_C, _C), jnp.bfloat16),
            pltpu.VMEM((_C, _C), jnp.float32),
            pltpu.VMEM((1, _C), jnp.float32),
        ],
        compiler_params=pltpu.CompilerParams(
            dimension_semantics=("arbitrary",),
        ),
    )(g3, xt, mav_table, pc2)
    return jnp.reshape(out, ())
